# Initial kernel scaffold; baseline (speedup 1.0000x reference)
import jax
import jax.numpy as jnp
from jax.experimental import pallas as pl


def kernel(x, table, pos_embedding):
    raise NotImplementedError("write your pallas kernel here")


# speedup vs baseline: 1.1637x; 1.1637x over previous
import functools

import jax
import jax.numpy as jnp
from jax import lax
from jax.experimental import pallas as pl
from jax.experimental.pallas import tpu as pltpu
from jax.experimental.pallas import tpu_sc as plsc

VOCAB = 1000000
D = 32
B = 4096
L = 200
BL = B * L

NC, NS = 2, 16
NW = NC * NS
PER_W = BL // NW
CH = 800
NCH = PER_W // CH

_mesh = plsc.VectorSubcoreMesh(core_axis_name="c", subcore_axis_name="s")


@functools.partial(
    pl.kernel,
    mesh=_mesh,
    out_type=jax.ShapeDtypeStruct((BL, D), jnp.float32),
    compiler_params=pltpu.CompilerParams(use_tc_tiling_on_sc=False),
    scratch_types=[
        pltpu.VMEM((CH,), jnp.int32),
        pltpu.VMEM((CH, D), jnp.float32),
        pltpu.VMEM((L, D), jnp.float32),
        pltpu.SemaphoreType.DMA,
    ],
)
def _embed_sc(table_hbm, idx_hbm, pe_hbm, out_hbm, idx_v, rows_v, pe_v, sem):
    wid = lax.axis_index("s") * NC + lax.axis_index("c")
    base = wid * PER_W

    pltpu.sync_copy(pe_hbm, pe_v)

    def chunk_body(c, carry):
        off = base + c * CH
        pltpu.sync_copy(idx_hbm.at[pl.ds(off, CH)], idx_v)
        pltpu.async_copy(table_hbm.at[idx_v], rows_v, sem).wait()

        def add_body(r, carry2):
            p = lax.rem(r, L)
            rows_v[r, pl.ds(0, 16)] += pe_v[p, pl.ds(0, 16)]
            rows_v[r, pl.ds(16, 16)] += pe_v[p, pl.ds(16, 16)]
            return carry2

        lax.fori_loop(0, CH, add_body, 0)
        pltpu.sync_copy(rows_v, out_hbm.at[pl.ds(off, CH)])
        return carry

    lax.fori_loop(0, NCH, chunk_body, 0)


def kernel(x, table, pos_embedding):
    idx = x.reshape(-1).astype(jnp.int32)
    pe = pos_embedding[:L, :].astype(jnp.float32)
    out = _embed_sc(table, idx, pe)
    return out.reshape(B, L, D)


# speedup vs baseline: 1.4923x; 1.2824x over previous
import functools

import jax
import jax.numpy as jnp
from jax import lax
from jax.experimental import pallas as pl
from jax.experimental.pallas import tpu as pltpu
from jax.experimental.pallas import tpu_sc as plsc

VOCAB = 1000000
D = 32
B = 4096
L = 200
BL = B * L

NC, NS = 2, 16
NW = NC * NS
PER_W = BL // NW
CH = 400
NCH = PER_W // CH
NBUF = 4
ROWS_UNROLL = 16

_mesh = plsc.VectorSubcoreMesh(core_axis_name="c", subcore_axis_name="s")


@functools.partial(
    pl.kernel,
    mesh=_mesh,
    out_type=jax.ShapeDtypeStruct((BL, D), jnp.float32),
    compiler_params=pltpu.CompilerParams(use_tc_tiling_on_sc=False),
    scratch_types=[
        pltpu.VMEM((PER_W,), jnp.int32),
        pltpu.VMEM((CH, D), jnp.float32),
        pltpu.VMEM((CH, D), jnp.float32),
        pltpu.VMEM((CH, D), jnp.float32),
        pltpu.VMEM((CH, D), jnp.float32),
        pltpu.VMEM((CH, D), jnp.float32),
        pltpu.SemaphoreType.DMA,
        pltpu.SemaphoreType.DMA,
        pltpu.SemaphoreType.DMA,
        pltpu.SemaphoreType.DMA,
        pltpu.SemaphoreType.DMA,
        pltpu.SemaphoreType.DMA,
        pltpu.SemaphoreType.DMA,
        pltpu.SemaphoreType.DMA,
    ],
)
def _embed_sc(table_hbm, idx_hbm, pet_hbm, out_hbm,
              idx_v, rows0, rows1, rows2, rows3, pe_v,
              sg0, sg1, sg2, sg3, ss0, ss1, ss2, ss3):
    wid = lax.axis_index("s") * NC + lax.axis_index("c")
    base = wid * PER_W

    pltpu.sync_copy(idx_hbm.at[pl.ds(base, PER_W)], idx_v)
    pltpu.sync_copy(pet_hbm, pe_v)

    rows = (rows0, rows1, rows2, rows3)
    sg = (sg0, sg1, sg2, sg3)
    ss = (ss0, ss1, ss2, ss3)

    def start_gather(c, buf):
        pltpu.async_copy(
            table_hbm.at[idx_v.at[pl.ds(c * CH, CH)]], rows[buf], sg[buf])

    def wait_gather(c, buf):
        pltpu.make_async_copy(
            table_hbm.at[idx_v.at[pl.ds(c * CH, CH)]], rows[buf], sg[buf]
        ).wait()

    def start_scatter(c, buf):
        pltpu.async_copy(
            rows[buf], out_hbm.at[pl.ds(base + c * CH, CH)], ss[buf])

    def wait_scatter(c, buf):
        pltpu.make_async_copy(
            rows[buf], out_hbm.at[pl.ds(base + c * CH, CH)], ss[buf]
        ).wait()

    def add_pe(buf):
        def add_body(i, carry):
            r0 = i * ROWS_UNROLL
            for j in range(ROWS_UNROLL):
                r = r0 + j
                rows[buf][r, pl.ds(0, 16)] += pe_v[r, pl.ds(0, 16)]
                rows[buf][r, pl.ds(16, 16)] += pe_v[r, pl.ds(16, 16)]
            return carry

        lax.fori_loop(0, CH // ROWS_UNROLL, add_body, 0)

    for b in range(NBUF):
        start_gather(b, b)

    def quad_body(p, carry):
        c0 = p * NBUF
        for b in range(NBUF):
            c = c0 + b
            wait_gather(c, b)
            add_pe(b)
            start_scatter(c, b)
            nxt = c + NBUF - 1
            pb = (b - 1) % NBUF

            @pl.when(jnp.logical_and(c >= 1, nxt < NCH))
            def _():
                wait_scatter(c - 1, pb)
                start_gather(nxt, pb)

        return carry

    lax.fori_loop(0, NCH // NBUF, quad_body, 0)

    for b in range(NBUF):
        c = NCH - NBUF + b
        wait_scatter(c, c % NBUF)


def kernel(x, table, pos_embedding):
    idx = x.reshape(-1).astype(jnp.int32)
    pe_tile = jnp.tile(pos_embedding[:L, :].astype(jnp.float32), (CH // L, 1))
    out = _embed_sc(table, idx, pe_tile)
    return out.reshape(B, L, D)
